# baseline (speedup 1.0000x reference)
import jax
import jax.numpy as jnp
from jax.experimental import pallas as pl
from jax.experimental.pallas import tpu as pltpu

_B = 1024
_D_IN = 64
_H = 1024
_D_OUT = 64
_K = 1024
_EDIM = 64
_N = _B * _D_IN


def _nt_dot(a, b):
    return jax.lax.dot_general(a, b, (((1,), (1,)), ((), ())),
                               preferred_element_type=jnp.float32)


def _fused_kernel(xr_ref, emb_ref, x_ref, w1_ref, b1_ref, wh_ref, bh_ref,
                  wo_ref, bo_ref, f_ref, loss_ref,
                  raw_ref, ind_ref, iota_ref, qlat_ref):
    j = pl.program_id(0)

    @pl.when(j == 0)
    def _init_iota():
        iota_ref[...] = jax.lax.broadcasted_iota(jnp.int32, (_K, 1024), 0)

    xb = xr_ref[0]
    xbb = jnp.broadcast_to(xb, (_EDIM + 1, 1024))
    e = jax.lax.broadcasted_iota(jnp.int32, (_EDIM + 1, 1024), 0)
    p = xbb
    acc = jnp.ones((_EDIM + 1, 1024), jnp.float32)
    for b in range(7):
        acc = jnp.where(((e >> b) & 1) == 1, acc * p, acc)
        if b < 6:
            p = p * p
    Paug = acc
    emb = emb_ref[...]
    embsq = jnp.sum(emb * emb, axis=1, keepdims=True)
    embaug = jnp.concatenate([embsq, emb * -2.0], axis=1)
    raw = jnp.dot(embaug, Paug, preferred_element_type=jnp.float32)
    raw_ref[pl.ds(jax.lax.rem(j, 2), 1)] = raw[None]
    part_rowsq = jnp.sum(Paug * Paug) - 1024.0

    vals = raw_ref[pl.ds(jax.lax.rem(j + 1, 2), 1)].reshape(_K, 1024)
    idxs = iota_ref[...]
    h = _K // 2
    while h >= 8:
        mask = vals[:h] <= vals[h:]
        vals = jnp.minimum(vals[:h], vals[h:])
        idxs = jnp.where(mask, idxs[:h], idxs[h:])
        h //= 2
    minv = jnp.min(vals, axis=0)
    amin = jnp.min(
        jnp.where(vals == minv[None, :], idxs, jnp.int32(1 << 30)),
        axis=0)
    jm = jax.lax.rem(j + _EDIM - 1, _EDIM)
    ind_ref[pl.ds(jm, 1), :] = amin.astype(jnp.float32)[None, :]

    qlat_ref[0, 0] = (jnp.where(j == 0, 0.0, qlat_ref[0, 0])
                      + jnp.where(j < _EDIM, part_rowsq, 0.0)
                      + jnp.where(j >= 1, jnp.sum(minv), 0.0))

    @pl.when(j == _EDIM)
    def _mlp_step():
        q = jnp.transpose(ind_ref[...])
        h1 = jnp.maximum(_nt_dot(q, w1_ref[...]) + b1_ref[...], 0.0)
        for _ in range(4):
            h1 = jnp.maximum(_nt_dot(h1, wh_ref[...]) + bh_ref[...], 0.0)
        f_ref[...] = jnp.maximum(_nt_dot(h1, wo_ref[...]) + bo_ref[...], 0.0)
        d = x_ref[...] - q
        e_sum = jnp.sum(d * d)
        loss_ref[...] = (qlat_ref[0, 0] * (1.0 / (_N * _EDIM))
                         + 0.25 * e_sum * (1.0 / _N)).reshape(1, 1)


def kernel(x, emb_w, W1, b1, Wh, bh, Wo, bo):
    xr3 = x.reshape(_EDIM, 1, 1024)
    last = _EDIM - 1

    const = lambda *blk: pl.BlockSpec(blk, lambda j: tuple(0 for _ in blk))
    f, loss = pl.pallas_call(
        _fused_kernel,
        grid=(_EDIM + 1,),
        in_specs=[
            pl.BlockSpec((1, 1, 1024), lambda j: (jnp.minimum(j, last), 0, 0)),
            const(_K, _EDIM),
            const(_B, _D_IN),
            const(_H, _D_IN),
            const(1, _H),
            const(_H, _H),
            const(1, _H),
            const(_D_OUT, _H),
            const(1, _D_OUT),
        ],
        out_specs=[
            const(_B, _D_OUT),
            const(1, 1),
        ],
        out_shape=[
            jax.ShapeDtypeStruct((_B, _D_OUT), jnp.float32),
            jax.ShapeDtypeStruct((1, 1), jnp.float32),
        ],
        scratch_shapes=[
            pltpu.VMEM((2, _K, 1024), jnp.float32),
            pltpu.VMEM((_EDIM, 1024), jnp.float32),
            pltpu.VMEM((_K, 1024), jnp.int32),
            pltpu.SMEM((1, 1), jnp.float32),
        ],
    )(xr3, emb_w, x, W1, b1.reshape(1, _H), Wh, bh.reshape(1, _H),
      Wo, bo.reshape(1, _D_OUT))

    return f, loss[0, 0]


# speedup vs baseline: 1.3277x; 1.3277x over previous
import jax
import jax.numpy as jnp
from jax.experimental import pallas as pl
from jax.experimental.pallas import tpu as pltpu

_B = 1024
_D_IN = 64
_H = 1024
_D_OUT = 64
_K = 1024
_EDIM = 64
_N = _B * _D_IN


def _nt_dot(a, b):
    return jax.lax.dot_general(a, b, (((1,), (1,)), ((), ())),
                               preferred_element_type=jnp.float32)


def _fused_kernel(xr_ref, emb_ref, x_ref, w1_ref, b1_ref, wh_ref, bh_ref,
                  wo_ref, bo_ref, f_ref, loss_ref,
                  raw_ref, sm_ref, ind_ref, qlat_ref):
    j = pl.program_id(0)
    p = jax.lax.rem(j, 2)
    pm = jax.lax.rem(j + 1, 2)

    xb = xr_ref[0]
    xbb = jnp.broadcast_to(xb, (_EDIM, 1024))
    e = jax.lax.broadcasted_iota(jnp.int32, (_EDIM, 1024), 0) + 1
    pw = xbb
    acc = jnp.ones((_EDIM, 1024), jnp.float32)
    for b in range(7):
        acc = jnp.where(((e >> b) & 1) == 1, acc * pw, acc)
        if b < 6:
            pw = pw * pw
    P = acc
    raw = jnp.dot(emb_ref[...] * -2.0, P,
                  preferred_element_type=jnp.float32)
    raw_ref[pl.ds(p, 1)] = raw[None]
    sm = jnp.sum(P, axis=0, keepdims=True)
    sm_ref[pl.ds(p, 1)] = jnp.broadcast_to(sm, (8, 1024))[None]
    part_rowsq = jnp.sum(P * P)

    emb = emb_ref[...]
    embsq = jnp.sum(emb * emb, axis=1, keepdims=True)
    smp = sm_ref[pl.ds(pm, 1)][0, 0:1, :]
    rawp = raw_ref[pl.ds(pm, 1)].reshape(_K, 1024)
    vals = (smp + embsq) + rawp
    idxs = jax.lax.broadcasted_iota(jnp.int32, (_K, 1024), 0)
    h = _K // 2
    while h >= 8:
        mask = vals[:h] <= vals[h:]
        vals = jnp.minimum(vals[:h], vals[h:])
        idxs = jnp.where(mask, idxs[:h], idxs[h:])
        h //= 2
    minv = jnp.min(vals, axis=0)
    amin = jnp.min(
        jnp.where(vals == minv[None, :], idxs, jnp.int32(1 << 30)),
        axis=0)
    jm = jax.lax.rem(j + _EDIM - 1, _EDIM)
    ind_ref[pl.ds(jm, 1), :] = amin.astype(jnp.float32)[None, :]
    part_min = jnp.sum(minv - smp[0])

    qlat_ref[0, 0] = (jnp.where(j == 0, 0.0, qlat_ref[0, 0])
                      + jnp.where(j < _EDIM, part_rowsq, 0.0)
                      + jnp.where(j >= 1, part_min, 0.0))

    @pl.when(j == _EDIM)
    def _mlp_step():
        q = jnp.transpose(ind_ref[...])
        h1 = jnp.maximum(_nt_dot(q, w1_ref[...]) + b1_ref[...], 0.0)
        for _ in range(4):
            h1 = jnp.maximum(_nt_dot(h1, wh_ref[...]) + bh_ref[...], 0.0)
        f_ref[...] = jnp.maximum(_nt_dot(h1, wo_ref[...]) + bo_ref[...], 0.0)
        d = x_ref[...] - q
        e_sum = jnp.sum(d * d)
        loss_ref[...] = (qlat_ref[0, 0] * (1.0 / (_N * _EDIM))
                         + 0.25 * e_sum * (1.0 / _N)).reshape(1, 1)


def kernel(x, emb_w, W1, b1, Wh, bh, Wo, bo):
    xr3 = x.reshape(_EDIM, 1, 1024)
    last = _EDIM - 1

    const = lambda *blk: pl.BlockSpec(blk, lambda j: tuple(0 for _ in blk))
    f, loss = pl.pallas_call(
        _fused_kernel,
        grid=(_EDIM + 1,),
        in_specs=[
            pl.BlockSpec((1, 1, 1024), lambda j: (jnp.minimum(j, last), 0, 0)),
            const(_K, _EDIM),
            const(_B, _D_IN),
            const(_H, _D_IN),
            const(1, _H),
            const(_H, _H),
            const(1, _H),
            const(_D_OUT, _H),
            const(1, _D_OUT),
        ],
        out_specs=[
            const(_B, _D_OUT),
            const(1, 1),
        ],
        out_shape=[
            jax.ShapeDtypeStruct((_B, _D_OUT), jnp.float32),
            jax.ShapeDtypeStruct((1, 1), jnp.float32),
        ],
        scratch_shapes=[
            pltpu.VMEM((2, _K, 1024), jnp.float32),
            pltpu.VMEM((2, 8, 1024), jnp.float32),
            pltpu.VMEM((_EDIM, 1024), jnp.float32),
            pltpu.SMEM((1, 1), jnp.float32),
        ],
    )(xr3, emb_w, x, W1, b1.reshape(1, _H), Wh, bh.reshape(1, _H),
      Wo, bo.reshape(1, _D_OUT))

    return f, loss[0, 0]


# speedup vs baseline: 2.2550x; 1.6984x over previous
import jax
import jax.numpy as jnp
from jax.experimental import pallas as pl
from jax.experimental.pallas import tpu as pltpu

_B = 1024
_D_IN = 64
_H = 1024
_D_OUT = 64
_K = 1024
_EDIM = 64
_N = _B * _D_IN
_T = 2
_W = 1024 * _T
_STEPS = _EDIM // _T


def _nt_dot(a, b):
    return jax.lax.dot_general(a, b, (((1,), (1,)), ((), ())),
                               preferred_element_type=jnp.float32)


def _fused_kernel(xr_ref, emb_ref, x_ref, w1_ref, b1_ref, wh_ref, bh_ref,
                  wo_ref, bo_ref, f_ref, loss_ref, ind_ref, qlat_ref):
    j = pl.program_id(0)

    @pl.when(j < _STEPS)
    def _dist_step():
        xb = xr_ref[0].reshape(1, _W)
        xbb = jnp.broadcast_to(xb, (_EDIM, _W))
        e = jax.lax.broadcasted_iota(jnp.int32, (_EDIM, _W), 0) + 1
        pw = xbb
        acc = jnp.ones((_EDIM, _W), jnp.float32)
        for b in range(7):
            acc = jnp.where(((e >> b) & 1) == 1, acc * pw, acc)
            if b < 6:
                pw = pw * pw
        P = acc
        emb = emb_ref[...]
        embsq = jnp.sum(emb * emb, axis=1, keepdims=True)
        sm = jnp.sum(P, axis=0, keepdims=True)
        dist = (sm + embsq) + jnp.dot(
            emb * -2.0, P, preferred_element_type=jnp.float32)
        vals = dist
        idxs = jax.lax.broadcasted_iota(jnp.int32, (_K, _W), 0)
        h = _K // 2
        while h >= 8:
            mask = vals[:h] <= vals[h:]
            vals = jnp.minimum(vals[:h], vals[h:])
            idxs = jnp.where(mask, idxs[:h], idxs[h:])
            h //= 2
        minv = jnp.min(vals, axis=0)
        amin = jnp.min(
            jnp.where(vals == minv[None, :], idxs, jnp.int32(1 << 30)),
            axis=0)
        ind_ref[pl.ds(j, 1)] = amin.astype(jnp.float32).reshape(1, _T, 1024)
        rowsq = jnp.sum(P * P, axis=0)
        part = jnp.sum(rowsq + (minv - sm[0]))

        @pl.when(j == 0)
        def _init():
            qlat_ref[0, 0] = 0.0

        qlat_ref[0, 0] += part

    @pl.when(j == _STEPS)
    def _mlp_step():
        q = jnp.transpose(ind_ref[...].reshape(_EDIM, 1024))
        h1 = jnp.maximum(_nt_dot(q, w1_ref[...]) + b1_ref[...], 0.0)
        for _ in range(4):
            h1 = jnp.maximum(_nt_dot(h1, wh_ref[...]) + bh_ref[...], 0.0)
        f_ref[...] = jnp.maximum(_nt_dot(h1, wo_ref[...]) + bo_ref[...], 0.0)
        d = x_ref[...] - q
        e_sum = jnp.sum(d * d)
        loss_ref[...] = (qlat_ref[0, 0] * (1.0 / (_N * _EDIM))
                         + 0.25 * e_sum * (1.0 / _N)).reshape(1, 1)


def kernel(x, emb_w, W1, b1, Wh, bh, Wo, bo):
    xr3 = x.reshape(_STEPS, _T, 1024)
    last = _STEPS - 1

    const = lambda *blk: pl.BlockSpec(blk, lambda j: tuple(0 for _ in blk))
    f, loss = pl.pallas_call(
        _fused_kernel,
        grid=(_STEPS + 1,),
        in_specs=[
            pl.BlockSpec((1, _T, 1024), lambda j: (jnp.minimum(j, last), 0, 0)),
            const(_K, _EDIM),
            const(_B, _D_IN),
            const(_H, _D_IN),
            const(1, _H),
            const(_H, _H),
            const(1, _H),
            const(_D_OUT, _H),
            const(1, _D_OUT),
        ],
        out_specs=[
            const(_B, _D_OUT),
            const(1, 1),
        ],
        out_shape=[
            jax.ShapeDtypeStruct((_B, _D_OUT), jnp.float32),
            jax.ShapeDtypeStruct((1, 1), jnp.float32),
        ],
        scratch_shapes=[
            pltpu.VMEM((_STEPS, _T, 1024), jnp.float32),
            pltpu.SMEM((1, 1), jnp.float32),
        ],
    )(xr3, emb_w, x, W1, b1.reshape(1, _H), Wh, bh.reshape(1, _H),
      Wo, bo.reshape(1, _D_OUT))

    return f, loss[0, 0]


# speedup vs baseline: 2.3260x; 1.0315x over previous
import jax
import jax.numpy as jnp
from jax.experimental import pallas as pl
from jax.experimental.pallas import tpu as pltpu

_B = 1024
_D_IN = 64
_H = 1024
_D_OUT = 64
_K = 1024
_EDIM = 64
_N = _B * _D_IN
_T = 4
_W = 1024 * _T
_STEPS = _EDIM // _T


def _nt_dot(a, b):
    return jax.lax.dot_general(a, b, (((1,), (1,)), ((), ())),
                               preferred_element_type=jnp.float32)


def _fused_kernel(xr_ref, emb_ref, x_ref, w1_ref, b1_ref, wh_ref, bh_ref,
                  wo_ref, bo_ref, f_ref, loss_ref, ind_ref, qlat_ref):
    j = pl.program_id(0)

    @pl.when(j < _STEPS)
    def _dist_step():
        xb = xr_ref[0].reshape(1, _W)
        xbb = jnp.broadcast_to(xb, (_EDIM, _W))
        e = jax.lax.broadcasted_iota(jnp.int32, (_EDIM, _W), 0) + 1
        pw = xbb
        acc = jnp.ones((_EDIM, _W), jnp.float32)
        for b in range(7):
            acc = jnp.where(((e >> b) & 1) == 1, acc * pw, acc)
            if b < 6:
                pw = pw * pw
        P = acc
        emb = emb_ref[...]
        embsq = jnp.sum(emb * emb, axis=1, keepdims=True)
        sm = jnp.sum(P, axis=0, keepdims=True)
        dist = (sm + embsq) + jnp.dot(
            emb * -2.0, P, preferred_element_type=jnp.float32)
        vals = dist
        idxs = jax.lax.broadcasted_iota(jnp.int32, (_K, _W), 0)
        h = _K // 2
        while h >= 8:
            mask = vals[:h] <= vals[h:]
            vals = jnp.minimum(vals[:h], vals[h:])
            idxs = jnp.where(mask, idxs[:h], idxs[h:])
            h //= 2
        minv = jnp.min(vals, axis=0)
        amin = jnp.min(
            jnp.where(vals == minv[None, :], idxs, jnp.int32(1 << 30)),
            axis=0)
        ind_ref[pl.ds(j, 1)] = amin.astype(jnp.float32).reshape(1, _T, 1024)
        rowsq = jnp.sum(P * P, axis=0)
        part = jnp.sum(rowsq + (minv - sm[0]))

        @pl.when(j == 0)
        def _init():
            qlat_ref[0, 0] = 0.0

        qlat_ref[0, 0] += part

    @pl.when(j == _STEPS)
    def _mlp_step():
        q = jnp.transpose(ind_ref[...].reshape(_EDIM, 1024))
        h1 = jnp.maximum(_nt_dot(q, w1_ref[...]) + b1_ref[...], 0.0)
        for _ in range(4):
            h1 = jnp.maximum(_nt_dot(h1, wh_ref[...]) + bh_ref[...], 0.0)
        f_ref[...] = jnp.maximum(_nt_dot(h1, wo_ref[...]) + bo_ref[...], 0.0)
        d = x_ref[...] - q
        e_sum = jnp.sum(d * d)
        loss_ref[...] = (qlat_ref[0, 0] * (1.0 / (_N * _EDIM))
                         + 0.25 * e_sum * (1.0 / _N)).reshape(1, 1)


def kernel(x, emb_w, W1, b1, Wh, bh, Wo, bo):
    xr3 = x.reshape(_STEPS, _T, 1024)
    last = _STEPS - 1

    const = lambda *blk: pl.BlockSpec(blk, lambda j: tuple(0 for _ in blk))
    f, loss = pl.pallas_call(
        _fused_kernel,
        grid=(_STEPS + 1,),
        in_specs=[
            pl.BlockSpec((1, _T, 1024), lambda j: (jnp.minimum(j, last), 0, 0)),
            const(_K, _EDIM),
            const(_B, _D_IN),
            const(_H, _D_IN),
            const(1, _H),
            const(_H, _H),
            const(1, _H),
            const(_D_OUT, _H),
            const(1, _D_OUT),
        ],
        out_specs=[
            const(_B, _D_OUT),
            const(1, 1),
        ],
        out_shape=[
            jax.ShapeDtypeStruct((_B, _D_OUT), jnp.float32),
            jax.ShapeDtypeStruct((1, 1), jnp.float32),
        ],
        scratch_shapes=[
            pltpu.VMEM((_STEPS, _T, 1024), jnp.float32),
            pltpu.SMEM((1, 1), jnp.float32),
        ],
    )(xr3, emb_w, x, W1, b1.reshape(1, _H), Wh, bh.reshape(1, _H),
      Wo, bo.reshape(1, _D_OUT))

    return f, loss[0, 0]
